# baseline (speedup 1.0000x reference)
import functools

import jax
import jax.numpy as jnp
from jax import lax
from jax.experimental import pallas as pl
from jax.experimental.pallas import tpu as pltpu
from jax.experimental.pallas import tpu_sc as plsc

VOCAB = 1000000
DIM = 64
B = 16384
NEG = 20

NC = 2
NS = 16
L = 16
NW = NC * NS
BPW = B // NW
C = 32
NCHUNK = BPW // C
NIDX_ROWS = C * NEG // 128
TROWS = VOCAB // 2


def _sc_dots(u_row, u_off, v_row, v_off, n_row, n_off, U2, V2):
    mesh = plsc.VectorSubcoreMesh(core_axis_name="c", subcore_axis_name="s")

    @functools.partial(
        pl.kernel,
        mesh=mesh,
        compiler_params=pltpu.CompilerParams(needs_layout_passes=False),
        out_type=[
            jax.ShapeDtypeStruct((B,), jnp.float32),
            jax.ShapeDtypeStruct((B,), jnp.float32),
        ],
        scratch_types=[
            pltpu.VMEM((NCHUNK, C), jnp.int32),
            pltpu.VMEM((NCHUNK, C), jnp.int32),
            pltpu.VMEM((NCHUNK, C), jnp.int32),
            pltpu.VMEM((NCHUNK, C), jnp.int32),
            pltpu.VMEM((NCHUNK * NIDX_ROWS, 128), jnp.int32),
            pltpu.VMEM((NCHUNK * NIDX_ROWS, 128), jnp.int32),
            pltpu.VMEM((C, 128), jnp.float32),
            pltpu.VMEM((C, 128), jnp.float32),
            pltpu.VMEM((C * NEG, 128), jnp.float32),
            pltpu.VMEM((C,), jnp.float32),
            pltpu.VMEM((C,), jnp.float32),
            pltpu.SemaphoreType.DMA,
        ],
    )
    def k(ur_hbm, uo_hbm, vr_hbm, vo_hbm, nr_hbm, no_hbm, u_hbm, v_hbm,
          p_hbm, q_hbm,
          urow, uoff, vrow, voff, nrow, noff, eu, ev, nrows, pc, qc, sem):
        wid = lax.axis_index("s") * NC + lax.axis_index("c")
        base = wid * BPW
        pltpu.sync_copy(ur_hbm.at[pl.ds(wid * NCHUNK, NCHUNK)], urow)
        pltpu.sync_copy(uo_hbm.at[pl.ds(wid * NCHUNK, NCHUNK)], uoff)
        pltpu.sync_copy(vr_hbm.at[pl.ds(wid * NCHUNK, NCHUNK)], vrow)
        pltpu.sync_copy(vo_hbm.at[pl.ds(wid * NCHUNK, NCHUNK)], voff)
        nb = NCHUNK * NIDX_ROWS
        pltpu.sync_copy(nr_hbm.at[pl.ds(wid * nb, nb)], nrow)
        pltpu.sync_copy(no_hbm.at[pl.ds(wid * nb, nb)], noff)

        lane = lax.iota(jnp.int32, L)

        def chunk_body(c, _):
            copies = [
                pltpu.async_copy(u_hbm.at[urow.at[c]], eu, sem),
                pltpu.async_copy(v_hbm.at[vrow.at[c]], ev, sem),
            ]
            for j in range(NIDX_ROWS):
                copies.append(pltpu.async_copy(
                    v_hbm.at[nrow.at[c * NIDX_ROWS + j]],
                    nrows.at[pl.ds(j * 128, 128)], sem))
            for cp in copies:
                cp.wait()

            for g in range(C // L):
                i_vec = lane + g * L
                ucol0 = uoff[c, pl.ds(g * L, L)]
                vcol0 = voff[c, pl.ds(g * L, L)]
                rvecs, ncol0 = [], []
                for n in range(NEG):
                    rv = i_vec * NEG + n
                    rvecs.append(rv)
                    ncol0.append(plsc.load_gather(
                        noff, [c * NIDX_ROWS + (rv >> 7), rv & 127]))

                def dim_body(d, carry):
                    p, q, ucol, vcol, *ncols = carry
                    val_u = plsc.load_gather(eu, [i_vec, ucol])
                    val_v = plsc.load_gather(ev, [i_vec, vcol])
                    p = p + val_u * val_v
                    new_ncols = []
                    for n in range(NEG):
                        val_n = plsc.load_gather(nrows, [rvecs[n], ncols[n]])
                        q = q + val_u * val_n
                        new_ncols.append(ncols[n] + 1)
                    return (p, q, ucol + 1, vcol + 1, *new_ncols)

                zero = jnp.zeros((L,), jnp.float32)
                out = lax.fori_loop(
                    0, DIM, dim_body,
                    (zero, zero, ucol0, vcol0, *ncol0), unroll=False)
                pc[pl.ds(g * L, L)] = out[0]
                qc[pl.ds(g * L, L)] = out[1]

            pltpu.sync_copy(pc, p_hbm.at[pl.ds(base + c * C, C)])
            pltpu.sync_copy(qc, q_hbm.at[pl.ds(base + c * C, C)])
            return 0

        lax.fori_loop(0, NCHUNK, chunk_body, 0, unroll=False)

    return k(u_row, u_off, v_row, v_off, n_row, n_off, U2, V2)


def _tc_loss_body(p_ref, q_ref, o_ref):
    a = p_ref[...]
    s = q_ref[...]

    def log_sigmoid(x):
        return jnp.minimum(x, 0.0) - jnp.log1p(jnp.exp(-jnp.abs(x)))

    ls = log_sigmoid(a) + log_sigmoid(-s)
    o_ref[...] = jnp.reshape(-jnp.sum(ls) / B, (1, 1))


def kernel(u_pos, v_pos, v_neg, U, V):
    u_pos = u_pos.astype(jnp.int32)
    v_pos = v_pos.astype(jnp.int32)
    v_neg = v_neg.astype(jnp.int32)
    u_row = (u_pos >> 1).reshape(B // C, C)
    u_off = ((u_pos & 1) << 6).reshape(B // C, C)
    v_row = (v_pos >> 1).reshape(B // C, C)
    v_off = ((v_pos & 1) << 6).reshape(B // C, C)
    n_flat = v_neg.reshape(-1)
    n_row = (n_flat >> 1).reshape(B * NEG // 128, 128)
    n_off = ((n_flat & 1) << 6).reshape(B * NEG // 128, 128)
    U2 = U.reshape(TROWS, 128)
    V2 = V.reshape(TROWS, 128)
    P, Q = _sc_dots(u_row, u_off, v_row, v_off, n_row, n_off, U2, V2)
    loss = pl.pallas_call(
        _tc_loss_body,
        out_shape=jax.ShapeDtypeStruct((1, 1), jnp.float32),
    )(P.reshape(128, 128), Q.reshape(128, 128))
    return loss[0, 0]


# speedup vs baseline: 1.0858x; 1.0858x over previous
import functools

import jax
import jax.numpy as jnp
from jax import lax
from jax.experimental import pallas as pl
from jax.experimental.pallas import tpu as pltpu
from jax.experimental.pallas import tpu_sc as plsc

VOCAB = 1000000
DIM = 64
B = 16384
NEG = 20

NC = 2
NS = 16
L = 16
NW = NC * NS
BPW = B // NW
C = 32
NCHUNK = BPW // C
NIDX_ROWS = C * NEG // 128

RPK_ROWS = 8000


def _repack_body(u_ref, v_ref, ou_ref, ov_ref):
    a = u_ref[...]
    ou_ref[:, 0:DIM] = a
    ou_ref[:, DIM:128] = a
    b = v_ref[...]
    ov_ref[:, 0:DIM] = b
    ov_ref[:, DIM:128] = b


def _repack(U, V):
    grid = VOCAB // RPK_ROWS
    spec_in = pl.BlockSpec((RPK_ROWS, DIM), lambda g: (g, 0))
    spec_out = pl.BlockSpec((RPK_ROWS, 128), lambda g: (g, 0))
    return pl.pallas_call(
        _repack_body,
        grid=(grid,),
        in_specs=[spec_in, spec_in],
        out_specs=[spec_out, spec_out],
        out_shape=[
            jax.ShapeDtypeStruct((VOCAB, 128), jnp.float32),
            jax.ShapeDtypeStruct((VOCAB, 128), jnp.float32),
        ],
    )(U, V)


def _sc_partials(u_pos2d, v_pos2d, v_neg2d, U2, V2):
    mesh = plsc.VectorSubcoreMesh(core_axis_name="c", subcore_axis_name="s")

    @functools.partial(
        pl.kernel,
        mesh=mesh,
        compiler_params=pltpu.CompilerParams(use_tc_tiling_on_sc=False),
        out_type=[
            jax.ShapeDtypeStruct((B, L), jnp.float32),
            jax.ShapeDtypeStruct((B, L), jnp.float32),
        ],
        scratch_types=[
            pltpu.VMEM((NCHUNK, C), jnp.int32),
            pltpu.VMEM((NCHUNK, C), jnp.int32),
            pltpu.VMEM((NCHUNK * NIDX_ROWS, 128), jnp.int32),
            pltpu.VMEM((C, 128), jnp.float32),
            pltpu.VMEM((C, 128), jnp.float32),
            pltpu.VMEM((C * NEG, 128), jnp.float32),
            pltpu.VMEM((BPW, L), jnp.float32),
            pltpu.VMEM((BPW, L), jnp.float32),
            pltpu.SemaphoreType.DMA,
        ],
    )
    def k(up_hbm, vp_hbm, vn_hbm, u_hbm, v_hbm, p_hbm, q_hbm,
          uidx, vidx, nidx, eu, ev, nrows, pw, qw, sem):
        wid = lax.axis_index("s") * NC + lax.axis_index("c")
        base = wid * BPW
        pltpu.sync_copy(up_hbm.at[pl.ds(wid * NCHUNK, NCHUNK)], uidx)
        pltpu.sync_copy(vp_hbm.at[pl.ds(wid * NCHUNK, NCHUNK)], vidx)
        nb = NCHUNK * NIDX_ROWS
        pltpu.sync_copy(vn_hbm.at[pl.ds(wid * nb, nb)], nidx)

        def chunk_body(c, _):
            copies = [
                pltpu.async_copy(u_hbm.at[uidx.at[c]], eu, sem),
                pltpu.async_copy(v_hbm.at[vidx.at[c]], ev, sem),
            ]
            for j in range(NIDX_ROWS):
                copies.append(pltpu.async_copy(
                    v_hbm.at[nidx.at[c * NIDX_ROWS + j]],
                    nrows.at[pl.ds(j * 128, 128)], sem))
            for cp in copies:
                cp.wait()

            def elem_body(i, _):
                e = [eu[i, pl.ds(kk * L, L)] for kk in range(4)]
                p = e[0] * ev[i, pl.ds(0, L)]
                for kk in range(1, 4):
                    p = p + e[kk] * ev[i, pl.ds(kk * L, L)]
                q = p - p
                for n in range(NEG):
                    r = i * NEG + n
                    for kk in range(4):
                        q = q + e[kk] * nrows[r, pl.ds(kk * L, L)]
                pw[c * C + i, pl.ds(0, L)] = p
                qw[c * C + i, pl.ds(0, L)] = q
                return 0

            lax.fori_loop(0, C, elem_body, 0, unroll=False)
            return 0

        lax.fori_loop(0, NCHUNK, chunk_body, 0, unroll=False)
        pltpu.sync_copy(pw, p_hbm.at[pl.ds(base, BPW)])
        pltpu.sync_copy(qw, q_hbm.at[pl.ds(base, BPW)])

    return k(u_pos2d, v_pos2d, v_neg2d, U2, V2)


def _tc_loss_body(p_ref, q_ref, o_ref):
    j = lax.broadcasted_iota(jnp.int32, (128, 8), 0)
    e = lax.broadcasted_iota(jnp.int32, (128, 8), 1)
    S = (j // L == e).astype(jnp.float32)
    a = jnp.dot(p_ref[...], S, preferred_element_type=jnp.float32)
    s = jnp.dot(q_ref[...], S, preferred_element_type=jnp.float32)

    def log_sigmoid(x):
        return jnp.minimum(x, 0.0) - jnp.log1p(jnp.exp(-jnp.abs(x)))

    ls = log_sigmoid(a) + log_sigmoid(-s)
    o_ref[...] = jnp.reshape(-jnp.sum(ls) / B, (1, 1))


def kernel(u_pos, v_pos, v_neg, U, V):
    u_pos2d = u_pos.astype(jnp.int32).reshape(B // C, C)
    v_pos2d = v_pos.astype(jnp.int32).reshape(B // C, C)
    v_neg2d = v_neg.astype(jnp.int32).reshape(B * NEG // 128, 128)
    U2, V2 = _repack(U, V)
    P, Q = _sc_partials(u_pos2d, v_pos2d, v_neg2d, U2, V2)
    loss = pl.pallas_call(
        _tc_loss_body,
        out_shape=jax.ShapeDtypeStruct((1, 1), jnp.float32),
    )(P.reshape(B * L // 128, 128), Q.reshape(B * L // 128, 128))
    return loss[0, 0]


# speedup vs baseline: 2.1815x; 2.0092x over previous
import functools

import jax
import jax.numpy as jnp
from jax import lax
from jax.experimental import pallas as pl
from jax.experimental.pallas import tpu as pltpu
from jax.experimental.pallas import tpu_sc as plsc

VOCAB = 1000000
DIM = 64
B = 16384
NEG = 20

NC = 2
NS = 16
L = 16
NW = NC * NS
BPW = B // NW
C = 32
NCHUNK = BPW // C
NIDX_ROWS = C * NEG // 128

RPK_ROWS = 8192


def _repack_body(u_ref, v_ref, ou_ref, ov_ref):
    a = u_ref[...].T
    ou_ref[:, 0:DIM] = a
    ou_ref[:, DIM:128] = a
    b = v_ref[...].T
    ov_ref[:, 0:DIM] = b
    ov_ref[:, DIM:128] = b


def _repack(UT, VT):
    grid = pl.cdiv(VOCAB, RPK_ROWS)
    spec_in = pl.BlockSpec((DIM, RPK_ROWS), lambda g: (0, g))
    spec_out = pl.BlockSpec((RPK_ROWS, 128), lambda g: (g, 0))
    return pl.pallas_call(
        _repack_body,
        grid=(grid,),
        in_specs=[spec_in, spec_in],
        out_specs=[spec_out, spec_out],
        out_shape=[
            jax.ShapeDtypeStruct((VOCAB, 128), jnp.float32),
            jax.ShapeDtypeStruct((VOCAB, 128), jnp.float32),
        ],
    )(UT, VT)


def _sc_partials(u_pos2d, v_pos2d, v_neg2d, U2, V2):
    mesh = plsc.VectorSubcoreMesh(core_axis_name="c", subcore_axis_name="s")

    @functools.partial(
        pl.kernel,
        mesh=mesh,
        compiler_params=pltpu.CompilerParams(use_tc_tiling_on_sc=False),
        out_type=[
            jax.ShapeDtypeStruct((B, L), jnp.float32),
            jax.ShapeDtypeStruct((B, L), jnp.float32),
        ],
        scratch_types=[
            pltpu.VMEM((NCHUNK, C), jnp.int32),
            pltpu.VMEM((NCHUNK, C), jnp.int32),
            pltpu.VMEM((NCHUNK * NIDX_ROWS, 128), jnp.int32),
            pltpu.VMEM((C, 128), jnp.float32),
            pltpu.VMEM((C, 128), jnp.float32),
            pltpu.VMEM((C * NEG, 128), jnp.float32),
            pltpu.VMEM((BPW, L), jnp.float32),
            pltpu.VMEM((BPW, L), jnp.float32),
            pltpu.SemaphoreType.DMA,
        ],
    )
    def k(up_hbm, vp_hbm, vn_hbm, u_hbm, v_hbm, p_hbm, q_hbm,
          uidx, vidx, nidx, eu, ev, nrows, pw, qw, sem):
        wid = lax.axis_index("s") * NC + lax.axis_index("c")
        base = wid * BPW
        pltpu.sync_copy(up_hbm.at[pl.ds(wid * NCHUNK, NCHUNK)], uidx)
        pltpu.sync_copy(vp_hbm.at[pl.ds(wid * NCHUNK, NCHUNK)], vidx)
        nb = NCHUNK * NIDX_ROWS
        pltpu.sync_copy(vn_hbm.at[pl.ds(wid * nb, nb)], nidx)

        def chunk_body(c, _):
            copies = [
                pltpu.async_copy(u_hbm.at[uidx.at[c]], eu, sem),
                pltpu.async_copy(v_hbm.at[vidx.at[c]], ev, sem),
            ]
            for j in range(NIDX_ROWS):
                copies.append(pltpu.async_copy(
                    v_hbm.at[nidx.at[c * NIDX_ROWS + j]],
                    nrows.at[pl.ds(j * 128, 128)], sem))
            for cp in copies:
                cp.wait()

            def elem_body(i, _):
                e = [eu[i, pl.ds(kk * L, L)] for kk in range(4)]
                p = e[0] * ev[i, pl.ds(0, L)]
                for kk in range(1, 4):
                    p = p + e[kk] * ev[i, pl.ds(kk * L, L)]
                q = p - p
                for n in range(NEG):
                    r = i * NEG + n
                    for kk in range(4):
                        q = q + e[kk] * nrows[r, pl.ds(kk * L, L)]
                pw[c * C + i, pl.ds(0, L)] = p
                qw[c * C + i, pl.ds(0, L)] = q
                return 0

            lax.fori_loop(0, C, elem_body, 0, unroll=False)
            return 0

        lax.fori_loop(0, NCHUNK, chunk_body, 0, unroll=False)
        pltpu.sync_copy(pw, p_hbm.at[pl.ds(base, BPW)])
        pltpu.sync_copy(qw, q_hbm.at[pl.ds(base, BPW)])

    return k(u_pos2d, v_pos2d, v_neg2d, U2, V2)


def _tc_loss_body(p_ref, q_ref, o_ref):
    j = lax.broadcasted_iota(jnp.int32, (128, 8), 0)
    e = lax.broadcasted_iota(jnp.int32, (128, 8), 1)
    S = (j // L == e).astype(jnp.float32)
    a = jnp.dot(p_ref[...], S, preferred_element_type=jnp.float32)
    s = jnp.dot(q_ref[...], S, preferred_element_type=jnp.float32)

    def log_sigmoid(x):
        return jnp.minimum(x, 0.0) - jnp.log1p(jnp.exp(-jnp.abs(x)))

    ls = log_sigmoid(a) + log_sigmoid(-s)
    o_ref[...] = jnp.reshape(-jnp.sum(ls) / B, (1, 1))


def kernel(u_pos, v_pos, v_neg, U, V):
    u_pos2d = u_pos.astype(jnp.int32).reshape(B // C, C)
    v_pos2d = v_pos.astype(jnp.int32).reshape(B // C, C)
    v_neg2d = v_neg.astype(jnp.int32).reshape(B * NEG // 128, 128)
    U2, V2 = _repack(U.T, V.T)
    P, Q = _sc_partials(u_pos2d, v_pos2d, v_neg2d, U2, V2)
    loss = pl.pallas_call(
        _tc_loss_body,
        out_shape=jax.ShapeDtypeStruct((1, 1), jnp.float32),
    )(P.reshape(B * L // 128, 128), Q.reshape(B * L // 128, 128))
    return loss[0, 0]


# speedup vs baseline: 2.4422x; 1.1195x over previous
import functools

import jax
import jax.numpy as jnp
from jax import lax
from jax.experimental import pallas as pl
from jax.experimental.pallas import tpu as pltpu
from jax.experimental.pallas import tpu_sc as plsc

VOCAB = 1000000
DIM = 64
B = 16384
NEG = 20

NC = 2
NS = 16
L = 16
NW = NC * NS
BPW = B // NW
C = 32
NCHUNK = BPW // C
NIDX_ROWS = C * NEG // 128

RPK_ROWS = 8192


def _repack_body(u_ref, v_ref, ou_ref, ov_ref):
    i0 = lax.broadcasted_iota(jnp.int32, (DIM, 128), 0)
    i1 = lax.broadcasted_iota(jnp.int32, (DIM, 128), 1)
    w = (i1 % DIM == i0).astype(jnp.float32)
    dn = (((0,), (0,)), ((), ()))
    ou_ref[...] = lax.dot_general(u_ref[...], w, dn,
                                  preferred_element_type=jnp.float32)
    ov_ref[...] = lax.dot_general(v_ref[...], w, dn,
                                  preferred_element_type=jnp.float32)


def _repack(UT, VT):
    grid = pl.cdiv(VOCAB, RPK_ROWS)
    spec_in = pl.BlockSpec((DIM, RPK_ROWS), lambda g: (0, g))
    spec_out = pl.BlockSpec((RPK_ROWS, 128), lambda g: (g, 0))
    return pl.pallas_call(
        _repack_body,
        grid=(grid,),
        in_specs=[spec_in, spec_in],
        out_specs=[spec_out, spec_out],
        out_shape=[
            jax.ShapeDtypeStruct((VOCAB, 128), jnp.float32),
            jax.ShapeDtypeStruct((VOCAB, 128), jnp.float32),
        ],
    )(UT, VT)


def _sc_partials(u_pos2d, v_pos2d, v_neg2d, U2, V2):
    mesh = plsc.VectorSubcoreMesh(core_axis_name="c", subcore_axis_name="s")

    @functools.partial(
        pl.kernel,
        mesh=mesh,
        compiler_params=pltpu.CompilerParams(use_tc_tiling_on_sc=False),
        out_type=[
            jax.ShapeDtypeStruct((B, L), jnp.float32),
            jax.ShapeDtypeStruct((B, L), jnp.float32),
        ],
        scratch_types=[
            pltpu.VMEM((NCHUNK, C), jnp.int32),
            pltpu.VMEM((NCHUNK, C), jnp.int32),
            pltpu.VMEM((NCHUNK * NIDX_ROWS, 128), jnp.int32),
            pltpu.VMEM((C, 128), jnp.float32),
            pltpu.VMEM((C, 128), jnp.float32),
            pltpu.VMEM((C * NEG, 128), jnp.float32),
            pltpu.VMEM((BPW, L), jnp.float32),
            pltpu.VMEM((BPW, L), jnp.float32),
            pltpu.SemaphoreType.DMA,
        ],
    )
    def k(up_hbm, vp_hbm, vn_hbm, u_hbm, v_hbm, p_hbm, q_hbm,
          uidx, vidx, nidx, eu, ev, nrows, pw, qw, sem):
        wid = lax.axis_index("s") * NC + lax.axis_index("c")
        base = wid * BPW
        pltpu.sync_copy(up_hbm.at[pl.ds(wid * NCHUNK, NCHUNK)], uidx)
        pltpu.sync_copy(vp_hbm.at[pl.ds(wid * NCHUNK, NCHUNK)], vidx)
        nb = NCHUNK * NIDX_ROWS
        pltpu.sync_copy(vn_hbm.at[pl.ds(wid * nb, nb)], nidx)

        def chunk_body(c, _):
            copies = [
                pltpu.async_copy(u_hbm.at[uidx.at[c]], eu, sem),
                pltpu.async_copy(v_hbm.at[vidx.at[c]], ev, sem),
            ]
            for j in range(NIDX_ROWS):
                copies.append(pltpu.async_copy(
                    v_hbm.at[nidx.at[c * NIDX_ROWS + j]],
                    nrows.at[pl.ds(j * 128, 128)], sem))
            for cp in copies:
                cp.wait()

            def elem_body(i, _):
                e = [eu[i, pl.ds(kk * L, L)] for kk in range(4)]
                p = e[0] * ev[i, pl.ds(0, L)]
                for kk in range(1, 4):
                    p = p + e[kk] * ev[i, pl.ds(kk * L, L)]
                q = p - p
                for n in range(NEG):
                    r = i * NEG + n
                    for kk in range(4):
                        q = q + e[kk] * nrows[r, pl.ds(kk * L, L)]
                pw[c * C + i, pl.ds(0, L)] = p
                qw[c * C + i, pl.ds(0, L)] = q
                return 0

            lax.fori_loop(0, C, elem_body, 0, unroll=False)
            return 0

        lax.fori_loop(0, NCHUNK, chunk_body, 0, unroll=False)
        pltpu.sync_copy(pw, p_hbm.at[pl.ds(base, BPW)])
        pltpu.sync_copy(qw, q_hbm.at[pl.ds(base, BPW)])

    return k(u_pos2d, v_pos2d, v_neg2d, U2, V2)


def _tc_loss_body(p_ref, q_ref, o_ref):
    j = lax.broadcasted_iota(jnp.int32, (128, 8), 0)
    e = lax.broadcasted_iota(jnp.int32, (128, 8), 1)
    S = (j // L == e).astype(jnp.float32)
    a = jnp.dot(p_ref[...], S, preferred_element_type=jnp.float32)
    s = jnp.dot(q_ref[...], S, preferred_element_type=jnp.float32)

    def log_sigmoid(x):
        return jnp.minimum(x, 0.0) - jnp.log1p(jnp.exp(-jnp.abs(x)))

    ls = log_sigmoid(a) + log_sigmoid(-s)
    o_ref[...] = jnp.reshape(-jnp.sum(ls) / B, (1, 1))


def kernel(u_pos, v_pos, v_neg, U, V):
    u_pos2d = u_pos.astype(jnp.int32).reshape(B // C, C)
    v_pos2d = v_pos.astype(jnp.int32).reshape(B // C, C)
    v_neg2d = v_neg.astype(jnp.int32).reshape(B * NEG // 128, 128)
    U2, V2 = _repack(U.T, V.T)
    P, Q = _sc_partials(u_pos2d, v_pos2d, v_neg2d, U2, V2)
    loss = pl.pallas_call(
        _tc_loss_body,
        out_shape=jax.ShapeDtypeStruct((1, 1), jnp.float32),
    )(P.reshape(B * L // 128, 128), Q.reshape(B * L // 128, 128))
    return loss[0, 0]


# speedup vs baseline: 2.6891x; 1.1011x over previous
import functools

import jax
import jax.numpy as jnp
from jax import lax
from jax.experimental import pallas as pl
from jax.experimental.pallas import tpu as pltpu
from jax.experimental.pallas import tpu_sc as plsc

VOCAB = 1000000
DIM = 64
B = 16384
NEG = 20

NC = 2
NS = 16
L = 16
NW = NC * NS
BPW = B // NW
C = 32
NCHUNK = BPW // C
NIDX_ROWS = C * NEG // 128

HALF = 1 << 19
RPK_ROWS = 4096
RPK_GRID = HALF // RPK_ROWS


def _repack_body(u1_ref, u2_ref, v1_ref, v2_ref, ou_ref, ov_ref):
    i0 = lax.broadcasted_iota(jnp.int32, (DIM, 128), 0)
    i1 = lax.broadcasted_iota(jnp.int32, (DIM, 128), 1)
    w1 = (i1 == i0).astype(jnp.float32)
    w2 = (i1 == i0 + DIM).astype(jnp.float32)
    dn = (((0,), (0,)), ((), ()))
    ou_ref[...] = (
        lax.dot_general(u1_ref[...], w1, dn,
                        preferred_element_type=jnp.float32)
        + lax.dot_general(u2_ref[...], w2, dn,
                          preferred_element_type=jnp.float32))
    ov_ref[...] = (
        lax.dot_general(v1_ref[...], w1, dn,
                        preferred_element_type=jnp.float32)
        + lax.dot_general(v2_ref[...], w2, dn,
                          preferred_element_type=jnp.float32))


def _repack(UT, VT):
    nblk = pl.cdiv(VOCAB, RPK_ROWS)
    lo = pl.BlockSpec((DIM, RPK_ROWS), lambda g: (0, g))
    hi = pl.BlockSpec((DIM, RPK_ROWS),
                      lambda g: (0, jnp.minimum(g + RPK_GRID, nblk - 1)))
    spec_out = pl.BlockSpec((RPK_ROWS, 128), lambda g: (g, 0))
    return pl.pallas_call(
        _repack_body,
        grid=(RPK_GRID,),
        in_specs=[lo, hi, lo, hi],
        out_specs=[spec_out, spec_out],
        out_shape=[
            jax.ShapeDtypeStruct((HALF, 128), jnp.float32),
            jax.ShapeDtypeStruct((HALF, 128), jnp.float32),
        ],
    )(UT, UT, VT, VT)


def _sc_partials(u_row2d, v_row2d, n_row2d, u_off, v_off, n_off, U2, V2):
    mesh = plsc.VectorSubcoreMesh(core_axis_name="c", subcore_axis_name="s")

    @functools.partial(
        pl.kernel,
        mesh=mesh,
        compiler_params=pltpu.CompilerParams(use_tc_tiling_on_sc=False),
        out_type=[
            jax.ShapeDtypeStruct((B, L), jnp.float32),
            jax.ShapeDtypeStruct((B, L), jnp.float32),
        ],
        scratch_types=[
            pltpu.VMEM((NCHUNK, C), jnp.int32),
            pltpu.VMEM((NCHUNK, C), jnp.int32),
            pltpu.VMEM((NCHUNK * NIDX_ROWS, 128), jnp.int32),
            pltpu.VMEM((BPW + L,), jnp.int32),
            pltpu.VMEM((BPW + L,), jnp.int32),
            pltpu.VMEM((BPW * NEG + L,), jnp.int32),
            pltpu.VMEM((C, 128), jnp.float32),
            pltpu.VMEM((C, 128), jnp.float32),
            pltpu.VMEM((C * NEG, 128), jnp.float32),
            pltpu.VMEM((BPW, L), jnp.float32),
            pltpu.VMEM((BPW, L), jnp.float32),
            pltpu.SemaphoreType.DMA,
        ],
    )
    def k(ur_hbm, vr_hbm, nr_hbm, uo_hbm, vo_hbm, no_hbm, u_hbm, v_hbm,
          p_hbm, q_hbm,
          urow, vrow, nrow, uoffv, voffv, noffv, eu, ev, nrows, pw, qw, sem):
        wid = lax.axis_index("s") * NC + lax.axis_index("c")
        base = wid * BPW
        pltpu.sync_copy(ur_hbm.at[pl.ds(wid * NCHUNK, NCHUNK)], urow)
        pltpu.sync_copy(vr_hbm.at[pl.ds(wid * NCHUNK, NCHUNK)], vrow)
        nb = NCHUNK * NIDX_ROWS
        pltpu.sync_copy(nr_hbm.at[pl.ds(wid * nb, nb)], nrow)
        pltpu.sync_copy(uo_hbm.at[pl.ds(base, BPW)], uoffv.at[pl.ds(0, BPW)])
        pltpu.sync_copy(vo_hbm.at[pl.ds(base, BPW)], voffv.at[pl.ds(0, BPW)])
        pltpu.sync_copy(no_hbm.at[pl.ds(base * NEG, BPW * NEG)],
                        noffv.at[pl.ds(0, BPW * NEG)])

        def chunk_body(c, _):
            copies = [
                pltpu.async_copy(u_hbm.at[urow.at[c]], eu, sem),
                pltpu.async_copy(v_hbm.at[vrow.at[c]], ev, sem),
            ]
            for j in range(NIDX_ROWS):
                copies.append(pltpu.async_copy(
                    v_hbm.at[nrow.at[c * NIDX_ROWS + j]],
                    nrows.at[pl.ds(j * 128, 128)], sem))
            for cp in copies:
                cp.wait()

            def elem_body(i, _):
                uo = uoffv[pl.ds(c * C + i, L)][0]
                vo = voffv[pl.ds(c * C + i, L)][0]
                e = [eu[i, pl.ds(uo + kk * L, L)] for kk in range(4)]
                p = e[0] * ev[i, pl.ds(vo, L)]
                for kk in range(1, 4):
                    p = p + e[kk] * ev[i, pl.ds(vo + kk * L, L)]
                q = p - p
                for n in range(NEG):
                    r = i * NEG + n
                    no = noffv[pl.ds(c * C * NEG + r, L)][0]
                    for kk in range(4):
                        q = q + e[kk] * nrows[r, pl.ds(no + kk * L, L)]
                pw[c * C + i, pl.ds(0, L)] = p
                qw[c * C + i, pl.ds(0, L)] = q
                return 0

            lax.fori_loop(0, C, elem_body, 0, unroll=False)
            return 0

        lax.fori_loop(0, NCHUNK, chunk_body, 0, unroll=False)
        pltpu.sync_copy(pw, p_hbm.at[pl.ds(base, BPW)])
        pltpu.sync_copy(qw, q_hbm.at[pl.ds(base, BPW)])

    return k(u_row2d, v_row2d, n_row2d, u_off, v_off, n_off, U2, V2)


def _tc_loss_body(p_ref, q_ref, o_ref):
    j = lax.broadcasted_iota(jnp.int32, (128, 8), 0)
    e = lax.broadcasted_iota(jnp.int32, (128, 8), 1)
    S = (j // L == e).astype(jnp.float32)
    a = jnp.dot(p_ref[...], S, preferred_element_type=jnp.float32)
    s = jnp.dot(q_ref[...], S, preferred_element_type=jnp.float32)

    def log_sigmoid(x):
        return jnp.minimum(x, 0.0) - jnp.log1p(jnp.exp(-jnp.abs(x)))

    ls = log_sigmoid(a) + log_sigmoid(-s)
    o_ref[...] = jnp.reshape(-jnp.sum(ls) / B, (1, 1))


def kernel(u_pos, v_pos, v_neg, U, V):
    u_pos = u_pos.astype(jnp.int32)
    v_pos = v_pos.astype(jnp.int32)
    n_flat = v_neg.astype(jnp.int32).reshape(-1)
    u_row2d = (u_pos & (HALF - 1)).reshape(B // C, C)
    v_row2d = (v_pos & (HALF - 1)).reshape(B // C, C)
    n_row2d = (n_flat & (HALF - 1)).reshape(B * NEG // 128, 128)
    u_off = (u_pos >> 19) << 6
    v_off = (v_pos >> 19) << 6
    n_off = (n_flat >> 19) << 6
    U2, V2 = _repack(U.T, V.T)
    P, Q = _sc_partials(u_row2d, v_row2d, n_row2d, u_off, v_off, n_off,
                        U2, V2)
    loss = pl.pallas_call(
        _tc_loss_body,
        out_shape=jax.ShapeDtypeStruct((1, 1), jnp.float32),
    )(P.reshape(B * L // 128, 128), Q.reshape(B * L // 128, 128))
    return loss[0, 0]
